# baseline (speedup 1.0000x reference)
import functools

import jax
import jax.numpy as jnp
from jax.experimental import pallas as pl
from jax.experimental.pallas import tpu as pltpu

_CHUNK = 400
_TAIL = (200, 104, 96)


def _gcn_kernel(
    x_hbm,
    a_hbm,
    w_ref,
    out_hbm,
    xv,
    support,
    abuf,
    obuf,
    x_sem,
    a_sem,
    o_sem,
    t_sem,
    to_sem,
    *,
    n_full,
):
    def a_copy(start, rows, b, boff, sem):
        return pltpu.make_async_copy(
            a_hbm.at[pl.ds(start, rows), :],
            abuf.at[b, pl.ds(boff, rows), :],
            sem,
        )

    a_copy(0, _CHUNK, 0, 0, a_sem.at[0]).start()
    a_copy(_CHUNK, _CHUNK, 1, 0, a_sem.at[1]).start()
    x_cp = pltpu.make_async_copy(x_hbm, xv, x_sem)
    x_cp.start()
    x_cp.wait()
    support[...] = jnp.dot(
        xv[...], w_ref[...], preferred_element_type=jnp.float32
    )

    t0 = n_full * _CHUNK
    t1 = t0 + _TAIL[0]
    t2 = t1 + _TAIL[1]

    def body(c, carry):
        b = jax.lax.rem(c, 2)
        a_copy(c * _CHUNK, _CHUNK, b, 0, a_sem.at[b]).wait()

        @pl.when(c >= 2)
        def _():
            pltpu.make_async_copy(
                obuf.at[b],
                out_hbm.at[pl.ds((c - 2) * _CHUNK, _CHUNK)],
                o_sem.at[b],
            ).wait()

        obuf[b] = jnp.dot(
            abuf[b], support[...], preferred_element_type=jnp.float32
        )
        pltpu.make_async_copy(
            obuf.at[b], out_hbm.at[pl.ds(c * _CHUNK, _CHUNK)], o_sem.at[b]
        ).start()

        @pl.when(c + 2 < n_full)
        def _():
            a_copy((c + 2) * _CHUNK, _CHUNK, b, 0, a_sem.at[b]).start()

        @pl.when(c == n_full - 2)
        def _():
            a_copy(t0, _TAIL[0], 0, 0, t_sem.at[0]).start()

        @pl.when(c == n_full - 1)
        def _():
            a_copy(t1, _TAIL[1], 1, 0, t_sem.at[1]).start()
            a_copy(t2, _TAIL[2], 1, _TAIL[1], t_sem.at[2]).start()

        return carry

    jax.lax.fori_loop(0, n_full, body, 0)

    pltpu.make_async_copy(
        obuf.at[0], out_hbm.at[pl.ds((n_full - 2) * _CHUNK, _CHUNK)], o_sem.at[0]
    ).wait()
    pltpu.make_async_copy(
        obuf.at[1], out_hbm.at[pl.ds((n_full - 1) * _CHUNK, _CHUNK)], o_sem.at[1]
    ).wait()

    a_copy(t0, _TAIL[0], 0, 0, t_sem.at[0]).wait()
    obuf[0, : _TAIL[0]] = jnp.dot(
        abuf[0, : _TAIL[0], :], support[...], preferred_element_type=jnp.float32
    )
    pltpu.make_async_copy(
        obuf.at[0, pl.ds(0, _TAIL[0])], out_hbm.at[pl.ds(t0, _TAIL[0])],
        to_sem.at[0],
    ).start()

    a_copy(t1, _TAIL[1], 1, 0, t_sem.at[1]).wait()
    obuf[1, : _TAIL[1]] = jnp.dot(
        abuf[1, : _TAIL[1], :], support[...], preferred_element_type=jnp.float32
    )
    pltpu.make_async_copy(
        obuf.at[1, pl.ds(0, _TAIL[1])], out_hbm.at[pl.ds(t1, _TAIL[1])],
        to_sem.at[1],
    ).start()

    a_copy(t2, _TAIL[2], 1, _TAIL[1], t_sem.at[2]).wait()
    obuf[1, _TAIL[1] : _TAIL[1] + _TAIL[2]] = jnp.dot(
        abuf[1, _TAIL[1] : _TAIL[1] + _TAIL[2], :],
        support[...],
        preferred_element_type=jnp.float32,
    )
    pltpu.make_async_copy(
        obuf.at[1, pl.ds(_TAIL[1], _TAIL[2])], out_hbm.at[pl.ds(t2, _TAIL[2])],
        to_sem.at[2],
    ).start()

    pltpu.make_async_copy(
        obuf.at[0, pl.ds(0, _TAIL[0])], out_hbm.at[pl.ds(t0, _TAIL[0])],
        to_sem.at[0],
    ).wait()
    pltpu.make_async_copy(
        obuf.at[1, pl.ds(0, _TAIL[1])], out_hbm.at[pl.ds(t1, _TAIL[1])],
        to_sem.at[1],
    ).wait()
    pltpu.make_async_copy(
        obuf.at[1, pl.ds(_TAIL[1], _TAIL[2])], out_hbm.at[pl.ds(t2, _TAIL[2])],
        to_sem.at[2],
    ).wait()


@functools.partial(jax.jit, static_argnames=())
def kernel(input, normed_A, weight):
    n, d_in = input.shape
    d_out = weight.shape[1]
    assert n == sum(_TAIL) + (n // _CHUNK - 1) * _CHUNK or n % _CHUNK == 0
    n_full = n // _CHUNK - 1

    return pl.pallas_call(
        functools.partial(_gcn_kernel, n_full=n_full),
        in_specs=[
            pl.BlockSpec(memory_space=pltpu.MemorySpace.HBM),
            pl.BlockSpec(memory_space=pltpu.MemorySpace.HBM),
            pl.BlockSpec(memory_space=pltpu.MemorySpace.VMEM),
        ],
        out_specs=pl.BlockSpec(memory_space=pltpu.MemorySpace.HBM),
        out_shape=jax.ShapeDtypeStruct((n, d_out), jnp.float32),
        scratch_shapes=[
            pltpu.VMEM((n, d_in), jnp.float32),
            pltpu.VMEM((n, d_out), jnp.float32),
            pltpu.VMEM((2, _CHUNK, n), jnp.float32),
            pltpu.VMEM((2, _CHUNK, d_out), jnp.float32),
            pltpu.SemaphoreType.DMA,
            pltpu.SemaphoreType.DMA((2,)),
            pltpu.SemaphoreType.DMA((2,)),
            pltpu.SemaphoreType.DMA((3,)),
            pltpu.SemaphoreType.DMA((3,)),
        ],
        compiler_params=pltpu.CompilerParams(
            vmem_limit_bytes=100 * 1024 * 1024,
        ),
    )(input, normed_A, weight)


# speedup vs baseline: 1.0445x; 1.0445x over previous
import functools

import jax
import jax.numpy as jnp
from jax.experimental import pallas as pl
from jax.experimental.pallas import tpu as pltpu


def _gcn_kernel(x_ref, a_ref, w_ref, out_ref, support_ref):
    @pl.when(pl.program_id(0) == 0)
    def _():
        support_ref[...] = jnp.dot(
            x_ref[...], w_ref[...], preferred_element_type=jnp.float32
        )

    out_ref[...] = jnp.dot(
        a_ref[...], support_ref[...], preferred_element_type=jnp.float32
    )


def _pick_block_m(n):
    best = None
    for b in range(8, 401, 8):
        if n % b == 0:
            best = b
    return best if best is not None else n


@functools.partial(jax.jit, static_argnames=())
def kernel(input, normed_A, weight):
    n, d_in = input.shape
    d_out = weight.shape[1]
    block_m = _pick_block_m(n)
    grid = (n // block_m,)

    return pl.pallas_call(
        _gcn_kernel,
        grid=grid,
        in_specs=[
            pl.BlockSpec((n, d_in), lambda i: (0, 0)),
            pl.BlockSpec((block_m, n), lambda i: (i, 0)),
            pl.BlockSpec((d_in, d_out), lambda i: (0, 0)),
        ],
        out_specs=pl.BlockSpec((block_m, d_out), lambda i: (i, 0)),
        out_shape=jax.ShapeDtypeStruct((n, d_out), jnp.float32),
        scratch_shapes=[pltpu.VMEM((n, d_out), jnp.float32)],
        compiler_params=pltpu.CompilerParams(
            dimension_semantics=("arbitrary",),
            vmem_limit_bytes=100 * 1024 * 1024,
        ),
    )(input, normed_A, weight)
